# baseline (speedup 1.0000x reference)
import dataclasses
import functools

import jax
import jax.numpy as jnp
from jax import lax
from jax.experimental import pallas as pl
from jax.experimental.pallas import tpu as pltpu
from jax.experimental.pallas import tpu_sc as plsc

NC = 2
NS = 16
LANES = 16
NW = NC * NS
CHUNK = 128


def _mesh():
    return plsc.VectorSubcoreMesh(core_axis_name="c", subcore_axis_name="s")


def _sc_params():
    cp = pltpu.CompilerParams()
    if "needs_layout_passes" in pltpu.CompilerParams.__dataclass_fields__:
        cp = dataclasses.replace(cp, needs_layout_passes=False)
    return cp


def _sc_deg(edata, n_pad):
    ec = edata.shape[1]
    rpt = n_pad // NS

    @functools.partial(
        pl.kernel,
        out_type=jax.ShapeDtypeStruct((NC, n_pad), jnp.float32),
        mesh=_mesh(),
        compiler_params=_sc_params(),
        scratch_types=[
            pltpu.VMEM((ec, 2, CHUNK), jnp.int32),
            pltpu.VMEM((n_pad,), jnp.float32),
            pltpu.VMEM((rpt,), jnp.float32),
            pltpu.VMEM((rpt,), jnp.float32),
            pltpu.VMEM_SHARED((NS, n_pad), jnp.float32),
        ],
    )
    def k(edata_hbm, out_hbm, ed_v, deg_v, acc_v, tmp_v, shared):
        cid = lax.axis_index("c")
        sid = lax.axis_index("s")
        wid = cid * NS + sid
        pltpu.sync_copy(edata_hbm.at[wid], ed_v)

        z16 = jnp.zeros((LANES,), jnp.float32)

        @pl.loop(0, n_pad // LANES)
        def _(i):
            deg_v[pl.ds(i * LANES, LANES)] = z16

        @pl.loop(0, ec)
        def _(j):
            for kk in range(CHUNK // LANES):
                idx = ed_v[j, 0, pl.ds(kk * LANES, LANES)]
                val = plsc.bitcast(ed_v[j, 1, pl.ds(kk * LANES, LANES)],
                                   jnp.float32)
                plsc.addupdate_scatter(deg_v, [idx], val)

        pltpu.sync_copy(deg_v, shared.at[sid])
        plsc.subcore_barrier()
        base = sid * rpt
        pltpu.sync_copy(shared.at[0, pl.ds(base, rpt)], acc_v)
        for t in range(1, NS):
            pltpu.sync_copy(shared.at[t, pl.ds(base, rpt)], tmp_v)

            @pl.loop(0, rpt // LANES)
            def _(i):
                sl = pl.ds(i * LANES, LANES)
                acc_v[sl] = acc_v[sl] + tmp_v[sl]

        pltpu.sync_copy(acc_v, out_hbm.at[cid, pl.ds(base, rpt)])

    return k(edata)


def _sc_agg(y, src_sh, edata, n_pad):
    ec = src_sh.shape[1]
    d = y.shape[1]
    rpt = n_pad // NS
    nblk = rpt // CHUNK

    @functools.partial(
        pl.kernel,
        out_type=jax.ShapeDtypeStruct((NC, n_pad, d), jnp.float32),
        mesh=_mesh(),
        compiler_params=_sc_params(),
        scratch_types=[
            pltpu.VMEM((ec, CHUNK), jnp.int32),
            pltpu.VMEM((ec, 2, CHUNK), jnp.int32),
            pltpu.VMEM((CHUNK, d), jnp.float32),
            pltpu.VMEM_SHARED((n_pad, d), jnp.float32),
            pltpu.SemaphoreType.DMA,
        ],
    )
    def k(y_hbm, src_hbm, edata_hbm, out_hbm,
          src_v, ed_v, rows_v, acc, sem):
        cid = lax.axis_index("c")
        sid = lax.axis_index("s")
        wid = cid * NS + sid
        pltpu.sync_copy(src_hbm.at[wid], src_v)
        pltpu.sync_copy(edata_hbm.at[wid], ed_v)

        z16 = jnp.zeros((LANES,), jnp.float32)

        @pl.loop(0, CHUNK)
        def _(r):
            for s in range(d // LANES):
                rows_v[r, pl.ds(s * LANES, LANES)] = z16

        base = sid * rpt
        for b in range(nblk):
            pltpu.sync_copy(rows_v, acc.at[pl.ds(base + b * CHUNK, CHUNK)])
        plsc.subcore_barrier()

        @pl.loop(0, ec)
        def _(j):
            pltpu.async_copy(y_hbm.at[src_v.at[j]], rows_v, sem).wait()

            @pl.loop(0, CHUNK)
            def _(r):
                jv = jnp.full((LANES,), j, jnp.int32)
                rv = jnp.full((LANES,), r, jnp.int32)
                one = jnp.full((LANES,), 1, jnp.int32)
                ewb = plsc.bitcast(plsc.load_gather(ed_v, [jv, one, rv]),
                                   jnp.float32)
                for s in range(d // LANES):
                    sl = pl.ds(s * LANES, LANES)
                    rows_v[r, sl] = rows_v[r, sl] * ewb

            pltpu.sync_copy(rows_v, acc.at[ed_v.at[j, 0]], add=True)

        plsc.subcore_barrier()
        for b in range(nblk):
            st = base + b * CHUNK
            pltpu.sync_copy(acc.at[pl.ds(st, CHUNK)],
                            out_hbm.at[cid, pl.ds(st, CHUNK)])

    return k(y, src_sh, edata)


_BLK = 512


def _dot(a, b):
    return jnp.dot(a, b, preferred_element_type=jnp.float32,
                   precision=lax.Precision.HIGHEST)


def _tc_first(x_pad, w1, degp_t):
    n_pad, d = x_pad.shape

    def body(x_ref, w_ref, dp_ref, y_ref, dinv_ref):
        deg = 1.0 + dp_ref[:, 0] + dp_ref[:, 1]
        dinv = jnp.where(deg > 0, lax.rsqrt(deg), 0.0)
        y_ref[...] = dinv[:, None] * _dot(x_ref[...], w_ref[...])
        dinv_ref[...] = dinv

    return pl.pallas_call(
        body,
        grid=(n_pad // _BLK,),
        in_specs=[
            pl.BlockSpec((_BLK, d), lambda i: (i, 0)),
            pl.BlockSpec((d, d), lambda i: (0, 0)),
            pl.BlockSpec((_BLK, 2), lambda i: (i, 0)),
        ],
        out_specs=[
            pl.BlockSpec((_BLK, d), lambda i: (i, 0)),
            pl.BlockSpec((_BLK,), lambda i: (i,)),
        ],
        out_shape=[
            jax.ShapeDtypeStruct((n_pad, d), jnp.float32),
            jax.ShapeDtypeStruct((n_pad,), jnp.float32),
        ],
    )(x_pad, w1, degp_t)


def _tc_mid(pagg, y_prev, dinv, b_prev, w_next):
    n_pad, d = y_prev.shape

    def body(p_ref, y_ref, dinv_ref, b_ref, w_ref, o_ref):
        dv = dinv_ref[...]
        t = p_ref[0] + p_ref[1] + y_ref[...]
        h = jnp.maximum(dv[:, None] * t + b_ref[...], 0.0)
        o_ref[...] = dv[:, None] * _dot(h, w_ref[...])

    return pl.pallas_call(
        body,
        grid=(n_pad // _BLK,),
        in_specs=[
            pl.BlockSpec((NC, _BLK, d), lambda i: (0, i, 0)),
            pl.BlockSpec((_BLK, d), lambda i: (i, 0)),
            pl.BlockSpec((_BLK,), lambda i: (i,)),
            pl.BlockSpec((d,), lambda i: (0,)),
            pl.BlockSpec((d, d), lambda i: (0, 0)),
        ],
        out_specs=pl.BlockSpec((_BLK, d), lambda i: (i, 0)),
        out_shape=jax.ShapeDtypeStruct((n_pad, d), jnp.float32),
    )(pagg, y_prev, dinv, b_prev, w_next)


def _tc_final(pagg, y_prev, dinv, b_prev, wl, bl):
    n_pad, d = y_prev.shape
    c = wl.shape[1]

    def body(p_ref, y_ref, dinv_ref, b_ref, wl_ref, bl_ref, o_ref):
        dv = dinv_ref[...]
        t = p_ref[0] + p_ref[1] + y_ref[...]
        h = jnp.maximum(dv[:, None] * t + b_ref[...], 0.0)
        o = _dot(h, wl_ref[...]) + bl_ref[...]
        m = jnp.max(o, axis=-1, keepdims=True)
        lse = jnp.log(jnp.sum(jnp.exp(o - m), axis=-1, keepdims=True)) + m
        o_ref[...] = o - lse

    return pl.pallas_call(
        body,
        grid=(n_pad // _BLK,),
        in_specs=[
            pl.BlockSpec((NC, _BLK, d), lambda i: (0, i, 0)),
            pl.BlockSpec((_BLK, d), lambda i: (i, 0)),
            pl.BlockSpec((_BLK,), lambda i: (i,)),
            pl.BlockSpec((d,), lambda i: (0,)),
            pl.BlockSpec((d, c), lambda i: (0, 0)),
            pl.BlockSpec((c,), lambda i: (0,)),
        ],
        out_specs=pl.BlockSpec((_BLK, c), lambda i: (i, 0)),
        out_shape=jax.ShapeDtypeStruct((n_pad, c), jnp.float32),
    )(pagg, y_prev, dinv, b_prev, wl, bl)


def kernel(x, edge_index, edge_weight, W1, b1, W2, b2, W3, b3, Wl, bl):
    n, d = x.shape
    e = edge_index.shape[1]
    ec = -(-e // (NW * CHUNK))
    ec = ec + (ec % 2)
    e_pad = NW * ec * CHUNK
    n_pad = -(-n // (NS * CHUNK)) * NS * CHUNK

    zpad_i = jnp.zeros((e_pad - e,), edge_index.dtype)
    zpad_f = jnp.zeros((e_pad - e,), edge_weight.dtype)
    src = jnp.concatenate([edge_index[0], zpad_i]).reshape(NW, ec, CHUNK)
    dst = jnp.concatenate([edge_index[1], zpad_i]).reshape(NW, ec, CHUNK)
    ew = jnp.concatenate([edge_weight, zpad_f]).reshape(NW, ec, CHUNK)
    ew_bits = lax.bitcast_convert_type(ew, jnp.int32)
    edata = jnp.stack([dst, ew_bits], axis=2)
    x_pad = jnp.pad(x, ((0, n_pad - n), (0, 0)))

    degp = _sc_deg(edata, n_pad)
    y1, dinv = _tc_first(x_pad, W1, degp.T)
    p1 = _sc_agg(y1, src, edata, n_pad)
    y2 = _tc_mid(p1, y1, dinv, b1, W2)
    p2 = _sc_agg(y2, src, edata, n_pad)
    y3 = _tc_mid(p2, y2, dinv, b2, W3)
    p3 = _sc_agg(y3, src, edata, n_pad)
    out = _tc_final(p3, y3, dinv, b3, Wl, bl)
    return out[:n]


# speedup vs baseline: 1.0839x; 1.0839x over previous
import dataclasses
import functools

import jax
import jax.numpy as jnp
import numpy as np
from jax import lax
from jax.experimental import pallas as pl
from jax.experimental.pallas import tpu as pltpu
from jax.experimental.pallas import tpu_sc as plsc

NC = 2
NS = 16
LANES = 16
NW = NC * NS
CHUNK = 128


def _mesh():
    return plsc.VectorSubcoreMesh(core_axis_name="c", subcore_axis_name="s")


def _sc_params():
    cp = pltpu.CompilerParams()
    fields = pltpu.CompilerParams.__dataclass_fields__
    if "needs_layout_passes" in fields:
        cp = dataclasses.replace(cp, needs_layout_passes=False)
    if "use_tc_tiling_on_sc" in fields:
        cp = dataclasses.replace(cp, use_tc_tiling_on_sc=False)
    return cp


def _sc_deg(edata, n_pad):
    ec = edata.shape[1]
    rpt = n_pad // NS

    @functools.partial(
        pl.kernel,
        out_type=jax.ShapeDtypeStruct((NC, n_pad), jnp.float32),
        mesh=_mesh(),
        compiler_params=_sc_params(),
        scratch_types=[
            pltpu.VMEM((ec, 2, CHUNK), jnp.int32),
            pltpu.VMEM((n_pad,), jnp.float32),
            pltpu.VMEM((rpt,), jnp.float32),
            pltpu.VMEM((rpt,), jnp.float32),
            pltpu.VMEM_SHARED((NS, n_pad), jnp.float32),
        ],
    )
    def k(edata_hbm, out_hbm, ed_v, deg_v, acc_v, tmp_v, shared):
        cid = lax.axis_index("c")
        sid = lax.axis_index("s")
        wid = cid * NS + sid
        pltpu.sync_copy(edata_hbm.at[wid], ed_v)

        z16 = jnp.zeros((LANES,), jnp.float32)

        @pl.loop(0, n_pad // LANES)
        def _(i):
            deg_v[pl.ds(i * LANES, LANES)] = z16

        @pl.loop(0, ec)
        def _(j):
            for kk in range(CHUNK // LANES):
                idx = ed_v[j, 0, pl.ds(kk * LANES, LANES)]
                val = plsc.bitcast(ed_v[j, 1, pl.ds(kk * LANES, LANES)],
                                   jnp.float32)
                plsc.addupdate_scatter(deg_v, [idx], val)

        pltpu.sync_copy(deg_v, shared.at[sid])
        plsc.subcore_barrier()
        base = sid * rpt
        pltpu.sync_copy(shared.at[0, pl.ds(base, rpt)], acc_v)
        for t in range(1, NS):
            pltpu.sync_copy(shared.at[t, pl.ds(base, rpt)], tmp_v)

            @pl.loop(0, rpt // LANES)
            def _(i):
                sl = pl.ds(i * LANES, LANES)
                acc_v[sl] = acc_v[sl] + tmp_v[sl]

        pltpu.sync_copy(acc_v, out_hbm.at[cid, pl.ds(base, rpt)])

    return k(edata)


def _sc_agg(y16i, src_sh, dst_sh, ew_sh, n_pad, d):
    ec = src_sh.shape[1]
    rpt = n_pad // NS
    nblk = rpt // CHUNK

    @functools.partial(
        pl.kernel,
        out_type=jax.ShapeDtypeStruct((NC, n_pad, d), jnp.float32),
        mesh=_mesh(),
        compiler_params=_sc_params(),
        scratch_types=[
            pltpu.VMEM((ec, CHUNK), jnp.int32),
            pltpu.VMEM((ec, CHUNK), jnp.int32),
            pltpu.VMEM((CHUNK,), jnp.float32),
            pltpu.VMEM((CHUNK,), jnp.float32),
            pltpu.VMEM((CHUNK, d // 2), jnp.int32),
            pltpu.VMEM((CHUNK, d), jnp.float32),
            pltpu.VMEM_SHARED((n_pad, d), jnp.float32),
            pltpu.SemaphoreType.DMA,
            pltpu.SemaphoreType.DMA,
            pltpu.SemaphoreType.DMA,
        ],
    )
    def k(y_hbm, src_hbm, dst_hbm, ew_hbm, out_hbm,
          src_v, dst_v, ew_a, ew_b, rows_v, sc_v, acc, sem, sem_ea, sem_eb):
        cid = lax.axis_index("c")
        sid = lax.axis_index("s")
        wid = cid * NS + sid
        pltpu.sync_copy(src_hbm.at[wid], src_v)
        pltpu.sync_copy(dst_hbm.at[wid], dst_v)

        z16 = jnp.zeros((LANES,), jnp.float32)

        @pl.loop(0, CHUNK)
        def _(r):
            for s in range(d // LANES):
                sc_v[r, pl.ds(s * LANES, LANES)] = z16

        base = sid * rpt
        for b in range(nblk):
            pltpu.sync_copy(sc_v, acc.at[pl.ds(base + b * CHUNK, CHUNK)])
        plsc.subcore_barrier()

        pltpu.async_copy(ew_hbm.at[wid, 0], ew_a, sem_ea)

        @pl.loop(0, ec, step=2)
        def _(j0):
            for half in range(2):
                j = j0 + half
                ew_cur = ew_a if half == 0 else ew_b
                ew_nxt = ew_b if half == 0 else ew_a
                se_cur = sem_ea if half == 0 else sem_eb
                se_nxt = sem_eb if half == 0 else sem_ea

                pltpu.sync_copy(y_hbm.at[src_v.at[j]], rows_v)

                @pl.when(j + 1 < ec)
                def _():
                    pltpu.async_copy(ew_hbm.at[wid, j + 1], ew_nxt, se_nxt)

                pltpu.make_async_copy(ew_hbm.at[wid, j], ew_cur, se_cur).wait()

                @pl.loop(0, CHUNK)
                def _(r):
                    rv = jnp.full((LANES,), r, jnp.int32)
                    ewb = plsc.load_gather(ew_cur, [rv])
                    for s in range(d // 32):
                        w = rows_v[r, pl.ds(s * LANES, LANES)]
                        lo = plsc.bitcast(w << 16, jnp.float32)
                        hi = plsc.bitcast(
                            w & jnp.int32(-65536), jnp.float32)
                        sc_v[r, pl.ds(s * 32, LANES)] = lo * ewb
                        sc_v[r, pl.ds(s * 32 + LANES, LANES)] = hi * ewb

                pltpu.sync_copy(sc_v, acc.at[dst_v.at[j]], add=True)

        plsc.subcore_barrier()
        for b in range(nblk):
            st = base + b * CHUNK
            pltpu.sync_copy(acc.at[pl.ds(st, CHUNK)],
                            out_hbm.at[cid, pl.ds(st, CHUNK)])

    return k(y16i, src_sh, dst_sh, ew_sh)


_BLK = 512


def _dot(a, b):
    return jnp.dot(a, b, preferred_element_type=jnp.float32,
                   precision=lax.Precision.HIGHEST)


def _tc_first(x_pad, w1, degp_t):
    n_pad, d = x_pad.shape

    def body(x_ref, w_ref, dp_ref, y_ref, dinv_ref):
        deg = 1.0 + dp_ref[:, 0] + dp_ref[:, 1]
        dinv = jnp.where(deg > 0, lax.rsqrt(deg), 0.0)
        y_ref[...] = dinv[:, None] * _dot(x_ref[...], w_ref[...])
        dinv_ref[...] = dinv

    return pl.pallas_call(
        body,
        grid=(n_pad // _BLK,),
        in_specs=[
            pl.BlockSpec((_BLK, d), lambda i: (i, 0)),
            pl.BlockSpec((d, d), lambda i: (0, 0)),
            pl.BlockSpec((_BLK, 2), lambda i: (i, 0)),
        ],
        out_specs=[
            pl.BlockSpec((_BLK, d), lambda i: (i, 0)),
            pl.BlockSpec((_BLK,), lambda i: (i,)),
        ],
        out_shape=[
            jax.ShapeDtypeStruct((n_pad, d), jnp.float32),
            jax.ShapeDtypeStruct((n_pad,), jnp.float32),
        ],
    )(x_pad, w1, degp_t)


def _tc_mid(pagg, y_prev, dinv, b_prev, w_next):
    n_pad, d = y_prev.shape

    def body(p_ref, y_ref, dinv_ref, b_ref, w_ref, o_ref):
        dv = dinv_ref[...]
        t = p_ref[0] + p_ref[1] + y_ref[...]
        h = jnp.maximum(dv[:, None] * t + b_ref[...], 0.0)
        o_ref[...] = dv[:, None] * _dot(h, w_ref[...])

    return pl.pallas_call(
        body,
        grid=(n_pad // _BLK,),
        in_specs=[
            pl.BlockSpec((NC, _BLK, d), lambda i: (0, i, 0)),
            pl.BlockSpec((_BLK, d), lambda i: (i, 0)),
            pl.BlockSpec((_BLK,), lambda i: (i,)),
            pl.BlockSpec((d,), lambda i: (0,)),
            pl.BlockSpec((d, d), lambda i: (0, 0)),
        ],
        out_specs=pl.BlockSpec((_BLK, d), lambda i: (i, 0)),
        out_shape=jax.ShapeDtypeStruct((n_pad, d), jnp.float32),
    )(pagg, y_prev, dinv, b_prev, w_next)


def _tc_final(pagg, y_prev, dinv, b_prev, wl, bl):
    n_pad, d = y_prev.shape
    c = wl.shape[1]

    def body(p_ref, y_ref, dinv_ref, b_ref, wl_ref, bl_ref, o_ref):
        dv = dinv_ref[...]
        t = p_ref[0] + p_ref[1] + y_ref[...]
        h = jnp.maximum(dv[:, None] * t + b_ref[...], 0.0)
        o = _dot(h, wl_ref[...]) + bl_ref[...]
        m = jnp.max(o, axis=-1, keepdims=True)
        lse = jnp.log(jnp.sum(jnp.exp(o - m), axis=-1, keepdims=True)) + m
        o_ref[...] = o - lse

    return pl.pallas_call(
        body,
        grid=(n_pad // _BLK,),
        in_specs=[
            pl.BlockSpec((NC, _BLK, d), lambda i: (0, i, 0)),
            pl.BlockSpec((_BLK, d), lambda i: (i, 0)),
            pl.BlockSpec((_BLK,), lambda i: (i,)),
            pl.BlockSpec((d,), lambda i: (0,)),
            pl.BlockSpec((d, c), lambda i: (0, 0)),
            pl.BlockSpec((c,), lambda i: (0,)),
        ],
        out_specs=pl.BlockSpec((_BLK, c), lambda i: (i, 0)),
        out_shape=jax.ShapeDtypeStruct((n_pad, c), jnp.float32),
    )(pagg, y_prev, dinv, b_prev, wl, bl)


def kernel(x, edge_index, edge_weight, W1, b1, W2, b2, W3, b3, Wl, bl):
    n, d = x.shape
    e = edge_index.shape[1]
    ec = -(-e // (NW * CHUNK))
    ec = ec + (ec % 2)
    e_pad = NW * ec * CHUNK
    n_pad = -(-n // (NS * CHUNK)) * NS * CHUNK

    zpad_i = jnp.zeros((e_pad - e,), edge_index.dtype)
    zpad_f = jnp.zeros((e_pad - e,), edge_weight.dtype)
    src = jnp.concatenate([edge_index[0], zpad_i]).reshape(NW, ec, CHUNK)
    dst = jnp.concatenate([edge_index[1], zpad_i]).reshape(NW, ec, CHUNK)
    ew = jnp.concatenate([edge_weight, zpad_f]).reshape(NW, ec, CHUNK)
    ew_bits = lax.bitcast_convert_type(ew, jnp.int32)
    edata = jnp.stack([dst, ew_bits], axis=2)
    x_pad = jnp.pad(x, ((0, n_pad - n), (0, 0)))

    h = W1.shape[1]
    perm = np.empty((h,), np.int32)
    for s in range(h // 32):
        for kk in range(16):
            perm[32 * s + 2 * kk] = 32 * s + kk
            perm[32 * s + 2 * kk + 1] = 32 * s + 16 + kk

    def pack_y(y):
        yb = y.astype(jnp.bfloat16)[:, perm]
        return lax.bitcast_convert_type(
            yb.reshape(n_pad, h // 2, 2), jnp.int32)

    degp = _sc_deg(edata, n_pad)
    y1, dinv = _tc_first(x_pad, W1, degp.T)
    p1 = _sc_agg(pack_y(y1), src, dst, ew, n_pad, h)
    y2 = _tc_mid(p1, y1, dinv, b1, W2)
    p2 = _sc_agg(pack_y(y2), src, dst, ew, n_pad, h)
    y3 = _tc_mid(p2, y2, dinv, b2, W3)
    p3 = _sc_agg(pack_y(y3), src, dst, ew, n_pad, h)
    out = _tc_final(p3, y3, dinv, b3, Wl, bl)
    return out[:n]


# speedup vs baseline: 1.4788x; 1.3643x over previous
import dataclasses
import functools

import jax
import jax.numpy as jnp
from jax import lax
from jax.experimental import pallas as pl
from jax.experimental.pallas import tpu as pltpu
from jax.experimental.pallas import tpu_sc as plsc

NC = 2
NS = 16
LANES = 16
NW = NC * NS
CHUNK = 128


def _mesh():
    return plsc.VectorSubcoreMesh(core_axis_name="c", subcore_axis_name="s")


def _sc_params():
    cp = pltpu.CompilerParams()
    fields = pltpu.CompilerParams.__dataclass_fields__
    if "needs_layout_passes" in fields:
        cp = dataclasses.replace(cp, needs_layout_passes=False)
    return cp


def _sc_deg(edata, n_pad):
    ec = edata.shape[1]
    rpt = n_pad // NS

    @functools.partial(
        pl.kernel,
        out_type=jax.ShapeDtypeStruct((NC, n_pad), jnp.float32),
        mesh=_mesh(),
        compiler_params=_sc_params(),
        scratch_types=[
            pltpu.VMEM((ec, 2, CHUNK), jnp.int32),
            pltpu.VMEM((n_pad,), jnp.float32),
            pltpu.VMEM((rpt,), jnp.float32),
            pltpu.VMEM((rpt,), jnp.float32),
            pltpu.VMEM_SHARED((NS, n_pad), jnp.float32),
        ],
    )
    def k(edata_hbm, out_hbm, ed_v, deg_v, acc_v, tmp_v, shared):
        cid = lax.axis_index("c")
        sid = lax.axis_index("s")
        wid = cid * NS + sid
        pltpu.sync_copy(edata_hbm.at[wid], ed_v)

        z16 = jnp.zeros((LANES,), jnp.float32)

        @pl.loop(0, n_pad // LANES)
        def _(i):
            deg_v[pl.ds(i * LANES, LANES)] = z16

        @pl.loop(0, ec)
        def _(j):
            for kk in range(CHUNK // LANES):
                idx = ed_v[j, 0, pl.ds(kk * LANES, LANES)]
                val = plsc.bitcast(ed_v[j, 1, pl.ds(kk * LANES, LANES)],
                                   jnp.float32)
                plsc.addupdate_scatter(deg_v, [idx], val)

        pltpu.sync_copy(deg_v, shared.at[sid])
        plsc.subcore_barrier()
        base = sid * rpt
        pltpu.sync_copy(shared.at[0, pl.ds(base, rpt)], acc_v)
        for t in range(1, NS):
            pltpu.sync_copy(shared.at[t, pl.ds(base, rpt)], tmp_v)

            @pl.loop(0, rpt // LANES)
            def _(i):
                sl = pl.ds(i * LANES, LANES)
                acc_v[sl] = acc_v[sl] + tmp_v[sl]

        pltpu.sync_copy(acc_v, out_hbm.at[cid, pl.ds(base, rpt)])

    return k(edata)


def _sc_agg(y, src_sh, dst_sh, ew_sh, n_pad, ec0, ec1):
    ec = src_sh.shape[1]
    d = y.shape[1]
    rpt = n_pad // NS
    nblk = rpt // CHUNK

    @functools.partial(
        pl.kernel,
        out_type=jax.ShapeDtypeStruct((NC, n_pad, d), jnp.float32),
        mesh=_mesh(),
        compiler_params=_sc_params(),
        scratch_types=[
            pltpu.VMEM((ec, CHUNK), jnp.int32),
            pltpu.VMEM((ec, CHUNK), jnp.int32),
            pltpu.VMEM((CHUNK,), jnp.float32),
            pltpu.VMEM((CHUNK,), jnp.float32),
            pltpu.VMEM((CHUNK, d), jnp.float32),
            pltpu.VMEM_SHARED((n_pad, d), jnp.float32),
            pltpu.SemaphoreType.DMA,
            pltpu.SemaphoreType.DMA,
            pltpu.SemaphoreType.DMA,
        ],
    )
    def k(y_hbm, src_hbm, dst_hbm, ew_hbm, out_hbm,
          src_v, dst_v, ew_a, ew_b, rows_v, acc, sem, sem_ea, sem_eb):
        cid = lax.axis_index("c")
        sid = lax.axis_index("s")
        wid = cid * NS + sid
        pltpu.sync_copy(src_hbm.at[wid], src_v)
        pltpu.sync_copy(dst_hbm.at[wid], dst_v)
        pltpu.async_copy(ew_hbm.at[wid, 0], ew_a, sem_ea)

        z16 = jnp.zeros((LANES,), jnp.float32)

        @pl.loop(0, CHUNK)
        def _(r):
            for s in range(d // LANES):
                rows_v[r, pl.ds(s * LANES, LANES)] = z16

        base = sid * rpt
        for b in range(nblk):
            pltpu.sync_copy(rows_v, acc.at[pl.ds(base + b * CHUNK, CHUNK)])
        plsc.subcore_barrier()

        ecb = jnp.where(cid == 0, ec0, ec1)

        @pl.loop(0, ecb, step=2)
        def _(j0):
            for half in range(2):
                j = j0 + half
                ew_cur = ew_a if half == 0 else ew_b
                ew_nxt = ew_b if half == 0 else ew_a
                se_cur = sem_ea if half == 0 else sem_eb
                se_nxt = sem_eb if half == 0 else sem_ea

                pltpu.async_copy(y_hbm.at[src_v.at[j]], rows_v, sem).wait()

                @pl.when(j + 1 < ecb)
                def _():
                    pltpu.async_copy(ew_hbm.at[wid, j + 1], ew_nxt, se_nxt)

                pltpu.make_async_copy(ew_hbm.at[wid, j], ew_cur, se_cur).wait()

                @pl.loop(0, CHUNK)
                def _(r):
                    rv = jnp.full((LANES,), r, jnp.int32)
                    ewb = plsc.load_gather(ew_cur, [rv])
                    for s in range(d // LANES):
                        sl = pl.ds(s * LANES, LANES)
                        rows_v[r, sl] = rows_v[r, sl] * ewb

                pltpu.sync_copy(rows_v, acc.at[dst_v.at[j]], add=True)

        plsc.subcore_barrier()
        for b in range(nblk):
            st = base + b * CHUNK
            pltpu.sync_copy(acc.at[pl.ds(st, CHUNK)],
                            out_hbm.at[cid, pl.ds(st, CHUNK)])

    return k(y, src_sh, dst_sh, ew_sh)


_BLK = 512


def _dot(a, b):
    return jnp.dot(a, b, preferred_element_type=jnp.float32,
                   precision=lax.Precision.HIGHEST)


def _tc_first(x_pad, w1, degp_t):
    n_pad, d = x_pad.shape

    def body(x_ref, w_ref, dp_ref, y_ref, dinv_ref):
        deg = 1.0 + dp_ref[:, 0] + dp_ref[:, 1]
        dinv = jnp.where(deg > 0, lax.rsqrt(deg), 0.0)
        y_ref[...] = dinv[:, None] * _dot(x_ref[...], w_ref[...])
        dinv_ref[...] = dinv

    return pl.pallas_call(
        body,
        grid=(n_pad // _BLK,),
        in_specs=[
            pl.BlockSpec((_BLK, d), lambda i: (i, 0)),
            pl.BlockSpec((d, d), lambda i: (0, 0)),
            pl.BlockSpec((_BLK, 2), lambda i: (i, 0)),
        ],
        out_specs=[
            pl.BlockSpec((_BLK, d), lambda i: (i, 0)),
            pl.BlockSpec((_BLK,), lambda i: (i,)),
        ],
        out_shape=[
            jax.ShapeDtypeStruct((n_pad, d), jnp.float32),
            jax.ShapeDtypeStruct((n_pad,), jnp.float32),
        ],
    )(x_pad, w1, degp_t)


def _tc_mid(pagg, y_prev, dinv, b_prev, w_next):
    n_pad, d = y_prev.shape

    def body(p_ref, y_ref, dinv_ref, b_ref, w_ref, o_ref):
        dv = dinv_ref[...]
        t = p_ref[0] + p_ref[1] + y_ref[...]
        h = jnp.maximum(dv[:, None] * t + b_ref[...], 0.0)
        o_ref[...] = dv[:, None] * _dot(h, w_ref[...])

    return pl.pallas_call(
        body,
        grid=(n_pad // _BLK,),
        in_specs=[
            pl.BlockSpec((NC, _BLK, d), lambda i: (0, i, 0)),
            pl.BlockSpec((_BLK, d), lambda i: (i, 0)),
            pl.BlockSpec((_BLK,), lambda i: (i,)),
            pl.BlockSpec((d,), lambda i: (0,)),
            pl.BlockSpec((d, d), lambda i: (0, 0)),
        ],
        out_specs=pl.BlockSpec((_BLK, d), lambda i: (i, 0)),
        out_shape=jax.ShapeDtypeStruct((n_pad, d), jnp.float32),
    )(pagg, y_prev, dinv, b_prev, w_next)


def _tc_final(pagg, y_prev, dinv, b_prev, wl, bl):
    n_pad, d = y_prev.shape
    c = wl.shape[1]

    def body(p_ref, y_ref, dinv_ref, b_ref, wl_ref, bl_ref, o_ref):
        dv = dinv_ref[...]
        t = p_ref[0] + p_ref[1] + y_ref[...]
        h = jnp.maximum(dv[:, None] * t + b_ref[...], 0.0)
        o = _dot(h, wl_ref[...]) + bl_ref[...]
        m = jnp.max(o, axis=-1, keepdims=True)
        lse = jnp.log(jnp.sum(jnp.exp(o - m), axis=-1, keepdims=True)) + m
        o_ref[...] = o - lse

    return pl.pallas_call(
        body,
        grid=(n_pad // _BLK,),
        in_specs=[
            pl.BlockSpec((NC, _BLK, d), lambda i: (0, i, 0)),
            pl.BlockSpec((_BLK, d), lambda i: (i, 0)),
            pl.BlockSpec((_BLK,), lambda i: (i,)),
            pl.BlockSpec((d,), lambda i: (0,)),
            pl.BlockSpec((d, c), lambda i: (0, 0)),
            pl.BlockSpec((c,), lambda i: (0,)),
        ],
        out_specs=pl.BlockSpec((_BLK, c), lambda i: (i, 0)),
        out_shape=jax.ShapeDtypeStruct((n_pad, c), jnp.float32),
    )(pagg, y_prev, dinv, b_prev, wl, bl)


def kernel(x, edge_index, edge_weight, W1, b1, W2, b2, W3, b3, Wl, bl):
    n, d = x.shape
    e = edge_index.shape[1]
    n_pad = -(-n // (NS * CHUNK)) * NS * CHUNK

    t = -(-e // (NS * CHUNK))
    ec0 = max(2, (int(t * 0.6) // 2) * 2)
    ec1 = -(-(t - ec0) // 2) * 2
    cap = NS * (ec0 + ec1) * CHUNK

    def shard(a):
        ap = jnp.concatenate([a, jnp.zeros((cap - e,), a.dtype)])
        c0 = ap[: NS * ec0 * CHUNK].reshape(NS, ec0, CHUNK)
        c1 = ap[NS * ec0 * CHUNK:].reshape(NS, ec1, CHUNK)
        c1 = jnp.pad(c1, ((0, 0), (0, ec0 - ec1), (0, 0)))
        return jnp.concatenate([c0, c1], axis=0)

    src = shard(edge_index[0])
    dst = shard(edge_index[1])
    ew = shard(edge_weight)
    ew_bits = lax.bitcast_convert_type(ew, jnp.int32)
    edata = jnp.stack([dst, ew_bits], axis=2)
    x_pad = jnp.pad(x, ((0, n_pad - n), (0, 0)))

    degp = _sc_deg(edata, n_pad)
    y1, dinv = _tc_first(x_pad, W1, degp.T)
    p1 = _sc_agg(y1, src, dst, ew, n_pad, ec0, ec1)
    y2 = _tc_mid(p1, y1, dinv, b1, W2)
    p2 = _sc_agg(y2, src, dst, ew, n_pad, ec0, ec1)
    y3 = _tc_mid(p2, y2, dinv, b2, W3)
    p3 = _sc_agg(y3, src, dst, ew, n_pad, ec0, ec1)
    out = _tc_final(p3, y3, dinv, b3, Wl, bl)
    return out[:n]


# speedup vs baseline: 1.4935x; 1.0100x over previous
import dataclasses
import functools

import jax
import jax.numpy as jnp
from jax import lax
from jax.experimental import pallas as pl
from jax.experimental.pallas import tpu as pltpu
from jax.experimental.pallas import tpu_sc as plsc

NC = 2
NS = 16
LANES = 16
NW = NC * NS
CHUNK = 128


def _mesh():
    return plsc.VectorSubcoreMesh(core_axis_name="c", subcore_axis_name="s")


def _sc_params():
    cp = pltpu.CompilerParams()
    fields = pltpu.CompilerParams.__dataclass_fields__
    if "needs_layout_passes" in fields:
        cp = dataclasses.replace(cp, needs_layout_passes=False)
    return cp


def _sc_deg(edata, n_pad):
    ec = edata.shape[1]
    rpt = n_pad // NS

    @functools.partial(
        pl.kernel,
        out_type=jax.ShapeDtypeStruct((NC, n_pad), jnp.float32),
        mesh=_mesh(),
        compiler_params=_sc_params(),
        scratch_types=[
            pltpu.VMEM((ec, 2, CHUNK), jnp.int32),
            pltpu.VMEM((n_pad,), jnp.float32),
            pltpu.VMEM((rpt,), jnp.float32),
            pltpu.VMEM((rpt,), jnp.float32),
            pltpu.VMEM_SHARED((NS, n_pad), jnp.float32),
        ],
    )
    def k(edata_hbm, out_hbm, ed_v, deg_v, acc_v, tmp_v, shared):
        cid = lax.axis_index("c")
        sid = lax.axis_index("s")
        wid = cid * NS + sid
        pltpu.sync_copy(edata_hbm.at[wid], ed_v)

        z16 = jnp.zeros((LANES,), jnp.float32)

        @pl.loop(0, n_pad // LANES)
        def _(i):
            deg_v[pl.ds(i * LANES, LANES)] = z16

        @pl.loop(0, ec)
        def _(j):
            for kk in range(CHUNK // LANES):
                idx = ed_v[j, 0, pl.ds(kk * LANES, LANES)]
                val = plsc.bitcast(ed_v[j, 1, pl.ds(kk * LANES, LANES)],
                                   jnp.float32)
                plsc.addupdate_scatter(deg_v, [idx], val)

        pltpu.sync_copy(deg_v, shared.at[sid])
        plsc.subcore_barrier()
        base = sid * rpt
        pltpu.sync_copy(shared.at[0, pl.ds(base, rpt)], acc_v)
        for t in range(1, NS):
            pltpu.sync_copy(shared.at[t, pl.ds(base, rpt)], tmp_v)

            @pl.loop(0, rpt // LANES)
            def _(i):
                sl = pl.ds(i * LANES, LANES)
                acc_v[sl] = acc_v[sl] + tmp_v[sl]

        pltpu.sync_copy(acc_v, out_hbm.at[cid, pl.ds(base, rpt)])

    return k(edata)


def _sc_agg(y, src_sh, dst_sh, ew_sh, n_pad, ec0, ec1):
    ec = src_sh.shape[1]
    d = y.shape[1]
    rpt = n_pad // NS
    nblk = rpt // CHUNK

    @functools.partial(
        pl.kernel,
        out_type=jax.ShapeDtypeStruct((NC, n_pad, d), jnp.float32),
        mesh=_mesh(),
        compiler_params=_sc_params(),
        scratch_types=[
            pltpu.VMEM((ec, CHUNK), jnp.int32),
            pltpu.VMEM((ec, CHUNK), jnp.int32),
            pltpu.VMEM((CHUNK,), jnp.float32),
            pltpu.VMEM((CHUNK,), jnp.float32),
            pltpu.VMEM((CHUNK, d), jnp.float32),
            pltpu.VMEM_SHARED((n_pad, d), jnp.float32),
            pltpu.SemaphoreType.DMA,
            pltpu.SemaphoreType.DMA,
            pltpu.SemaphoreType.DMA,
        ],
    )
    def k(y_hbm, src_hbm, dst_hbm, ew_hbm, out_hbm,
          src_v, dst_v, ew_a, ew_b, rows_v, acc, sem, sem_ea, sem_eb):
        cid = lax.axis_index("c")
        sid = lax.axis_index("s")
        wid = cid * NS + sid
        pltpu.sync_copy(src_hbm.at[wid], src_v)
        pltpu.sync_copy(dst_hbm.at[wid], dst_v)
        pltpu.async_copy(ew_hbm.at[wid, 0], ew_a, sem_ea)

        z16 = jnp.zeros((LANES,), jnp.float32)

        @pl.loop(0, CHUNK)
        def _(r):
            for s in range(d // LANES):
                rows_v[r, pl.ds(s * LANES, LANES)] = z16

        base = sid * rpt
        for b in range(nblk):
            pltpu.sync_copy(rows_v, acc.at[pl.ds(base + b * CHUNK, CHUNK)])
        plsc.subcore_barrier()

        ecb = jnp.where(cid == 0, ec0, ec1)

        @pl.loop(0, ecb, step=2)
        def _(j0):
            for half in range(2):
                j = j0 + half
                ew_cur = ew_a if half == 0 else ew_b
                ew_nxt = ew_b if half == 0 else ew_a
                se_cur = sem_ea if half == 0 else sem_eb
                se_nxt = sem_eb if half == 0 else sem_ea

                pltpu.async_copy(y_hbm.at[src_v.at[j]], rows_v, sem).wait()

                @pl.when(j + 1 < ecb)
                def _():
                    pltpu.async_copy(ew_hbm.at[wid, j + 1], ew_nxt, se_nxt)

                pltpu.make_async_copy(ew_hbm.at[wid, j], ew_cur, se_cur).wait()

                @pl.loop(0, CHUNK)
                def _(r):
                    rv = jnp.full((LANES,), r, jnp.int32)
                    ewb = plsc.load_gather(ew_cur, [rv])
                    for s in range(d // LANES):
                        sl = pl.ds(s * LANES, LANES)
                        rows_v[r, sl] = rows_v[r, sl] * ewb

                pltpu.sync_copy(rows_v, acc.at[dst_v.at[j]], add=True)

        plsc.subcore_barrier()
        for b in range(nblk):
            st = base + b * CHUNK
            pltpu.sync_copy(acc.at[pl.ds(st, CHUNK)],
                            out_hbm.at[cid, pl.ds(st, CHUNK)])

    return k(y, src_sh, dst_sh, ew_sh)


_BLK = 512


def _dot(a, b):
    return jnp.dot(a, b, preferred_element_type=jnp.float32,
                   precision=lax.Precision.HIGHEST)


def _tc_first(x_pad, w1, degp_t):
    n_pad, d = x_pad.shape

    def body(x_ref, w_ref, dp_ref, y_ref, dinv_ref):
        deg = 1.0 + dp_ref[:, 0] + dp_ref[:, 1]
        dinv = jnp.where(deg > 0, lax.rsqrt(deg), 0.0)
        y_ref[...] = dinv[:, None] * _dot(x_ref[...], w_ref[...])
        dinv_ref[...] = dinv

    return pl.pallas_call(
        body,
        grid=(n_pad // _BLK,),
        in_specs=[
            pl.BlockSpec((_BLK, d), lambda i: (i, 0)),
            pl.BlockSpec((d, d), lambda i: (0, 0)),
            pl.BlockSpec((_BLK, 2), lambda i: (i, 0)),
        ],
        out_specs=[
            pl.BlockSpec((_BLK, d), lambda i: (i, 0)),
            pl.BlockSpec((_BLK,), lambda i: (i,)),
        ],
        out_shape=[
            jax.ShapeDtypeStruct((n_pad, d), jnp.float32),
            jax.ShapeDtypeStruct((n_pad,), jnp.float32),
        ],
    )(x_pad, w1, degp_t)


def _tc_mid(pagg, y_prev, dinv, b_prev, w_next):
    n_pad, d = y_prev.shape

    def body(p_ref, y_ref, dinv_ref, b_ref, w_ref, o_ref):
        dv = dinv_ref[...]
        t = p_ref[0] + p_ref[1] + y_ref[...]
        h = jnp.maximum(dv[:, None] * t + b_ref[...], 0.0)
        o_ref[...] = dv[:, None] * _dot(h, w_ref[...])

    return pl.pallas_call(
        body,
        grid=(n_pad // _BLK,),
        in_specs=[
            pl.BlockSpec((NC, _BLK, d), lambda i: (0, i, 0)),
            pl.BlockSpec((_BLK, d), lambda i: (i, 0)),
            pl.BlockSpec((_BLK,), lambda i: (i,)),
            pl.BlockSpec((d,), lambda i: (0,)),
            pl.BlockSpec((d, d), lambda i: (0, 0)),
        ],
        out_specs=pl.BlockSpec((_BLK, d), lambda i: (i, 0)),
        out_shape=jax.ShapeDtypeStruct((n_pad, d), jnp.float32),
    )(pagg, y_prev, dinv, b_prev, w_next)


def _tc_final(pagg, y_prev, dinv, b_prev, wl, bl):
    n_pad, d = y_prev.shape
    c = wl.shape[1]

    def body(p_ref, y_ref, dinv_ref, b_ref, wl_ref, bl_ref, o_ref):
        dv = dinv_ref[...]
        t = p_ref[0] + p_ref[1] + y_ref[...]
        h = jnp.maximum(dv[:, None] * t + b_ref[...], 0.0)
        o = _dot(h, wl_ref[...]) + bl_ref[...]
        m = jnp.max(o, axis=-1, keepdims=True)
        lse = jnp.log(jnp.sum(jnp.exp(o - m), axis=-1, keepdims=True)) + m
        o_ref[...] = o - lse

    return pl.pallas_call(
        body,
        grid=(n_pad // _BLK,),
        in_specs=[
            pl.BlockSpec((NC, _BLK, d), lambda i: (0, i, 0)),
            pl.BlockSpec((_BLK, d), lambda i: (i, 0)),
            pl.BlockSpec((_BLK,), lambda i: (i,)),
            pl.BlockSpec((d,), lambda i: (0,)),
            pl.BlockSpec((d, c), lambda i: (0, 0)),
            pl.BlockSpec((c,), lambda i: (0,)),
        ],
        out_specs=pl.BlockSpec((_BLK, c), lambda i: (i, 0)),
        out_shape=jax.ShapeDtypeStruct((n_pad, c), jnp.float32),
    )(pagg, y_prev, dinv, b_prev, wl, bl)


def kernel(x, edge_index, edge_weight, W1, b1, W2, b2, W3, b3, Wl, bl):
    n, d = x.shape
    e = edge_index.shape[1]
    n_pad = -(-n // (NS * CHUNK)) * NS * CHUNK

    t = -(-e // (NS * CHUNK))
    ec0 = max(2, (int(t * 0.625) // 2) * 2)
    ec1 = -(-(t - ec0) // 2) * 2
    cap = NS * (ec0 + ec1) * CHUNK

    def shard(a):
        ap = jnp.concatenate([a, jnp.zeros((cap - e,), a.dtype)])
        c0 = ap[: NS * ec0 * CHUNK].reshape(NS, ec0, CHUNK)
        c1 = ap[NS * ec0 * CHUNK:].reshape(NS, ec1, CHUNK)
        c1 = jnp.pad(c1, ((0, 0), (0, ec0 - ec1), (0, 0)))
        return jnp.concatenate([c0, c1], axis=0)

    src = shard(edge_index[0])
    dst = shard(edge_index[1])
    ew = shard(edge_weight)
    ew_bits = lax.bitcast_convert_type(ew, jnp.int32)
    edata = jnp.stack([dst, ew_bits], axis=2)
    x_pad = jnp.pad(x, ((0, n_pad - n), (0, 0)))

    degp = _sc_deg(edata, n_pad)
    y1, dinv = _tc_first(x_pad, W1, degp.T)
    p1 = _sc_agg(y1, src, dst, ew, n_pad, ec0, ec1)
    y2 = _tc_mid(p1, y1, dinv, b1, W2)
    p2 = _sc_agg(y2, src, dst, ew, n_pad, ec0, ec1)
    y3 = _tc_mid(p2, y2, dinv, b2, W3)
    p3 = _sc_agg(y3, src, dst, ew, n_pad, ec0, ec1)
    out = _tc_final(p3, y3, dinv, b3, Wl, bl)
    return out[:n]


# speedup vs baseline: 1.5378x; 1.0296x over previous
import dataclasses
import functools

import jax
import jax.numpy as jnp
from jax import lax
from jax.experimental import pallas as pl
from jax.experimental.pallas import tpu as pltpu
from jax.experimental.pallas import tpu_sc as plsc

NC = 2
NS = 16
LANES = 16
NW = NC * NS
CHUNK = 128


def _mesh():
    return plsc.VectorSubcoreMesh(core_axis_name="c", subcore_axis_name="s")


def _sc_params():
    cp = pltpu.CompilerParams()
    fields = pltpu.CompilerParams.__dataclass_fields__
    if "needs_layout_passes" in fields:
        cp = dataclasses.replace(cp, needs_layout_passes=False)
    return cp


def _sc_deg(edata, n_pad):
    ec = edata.shape[1]
    rpt = n_pad // NS

    @functools.partial(
        pl.kernel,
        out_type=jax.ShapeDtypeStruct((NC, n_pad), jnp.float32),
        mesh=_mesh(),
        compiler_params=_sc_params(),
        scratch_types=[
            pltpu.VMEM((ec, 2, CHUNK), jnp.int32),
            pltpu.VMEM((n_pad,), jnp.float32),
            pltpu.VMEM((rpt,), jnp.float32),
            pltpu.VMEM((rpt,), jnp.float32),
            pltpu.VMEM_SHARED((NS, n_pad), jnp.float32),
        ],
    )
    def k(edata_hbm, out_hbm, ed_v, deg_v, acc_v, tmp_v, shared):
        cid = lax.axis_index("c")
        sid = lax.axis_index("s")
        wid = cid * NS + sid
        pltpu.sync_copy(edata_hbm.at[wid], ed_v)

        z16 = jnp.zeros((LANES,), jnp.float32)

        @pl.loop(0, n_pad // LANES)
        def _(i):
            deg_v[pl.ds(i * LANES, LANES)] = z16

        @pl.loop(0, ec)
        def _(j):
            for kk in range(CHUNK // LANES):
                idx = ed_v[j, 0, pl.ds(kk * LANES, LANES)]
                val = plsc.bitcast(ed_v[j, 1, pl.ds(kk * LANES, LANES)],
                                   jnp.float32)
                plsc.addupdate_scatter(deg_v, [idx], val)

        pltpu.sync_copy(deg_v, shared.at[sid])
        plsc.subcore_barrier()
        base = sid * rpt
        pltpu.sync_copy(shared.at[0, pl.ds(base, rpt)], acc_v)
        for t in range(1, NS):
            pltpu.sync_copy(shared.at[t, pl.ds(base, rpt)], tmp_v)

            @pl.loop(0, rpt // LANES)
            def _(i):
                sl = pl.ds(i * LANES, LANES)
                acc_v[sl] = acc_v[sl] + tmp_v[sl]

        pltpu.sync_copy(acc_v, out_hbm.at[cid, pl.ds(base, rpt)])

    return k(edata)


def _sc_agg(y, src_sh, dst_sh, ew_sh, n_pad, ec0, ec1):
    ec = src_sh.shape[1]
    d = y.shape[1]
    rpt = n_pad // NS
    nblk = rpt // CHUNK

    @functools.partial(
        pl.kernel,
        out_type=jax.ShapeDtypeStruct((NC, n_pad, d), jnp.float32),
        mesh=_mesh(),
        compiler_params=_sc_params(),
        scratch_types=[
            pltpu.VMEM((ec, CHUNK), jnp.int32),
            pltpu.VMEM((ec, CHUNK), jnp.int32),
            pltpu.VMEM((CHUNK,), jnp.float32),
            pltpu.VMEM((CHUNK,), jnp.float32),
            pltpu.VMEM((CHUNK, d), jnp.float32),
            pltpu.VMEM_SHARED((n_pad, d), jnp.float32),
            pltpu.SemaphoreType.DMA,
            pltpu.SemaphoreType.DMA,
            pltpu.SemaphoreType.DMA,
        ],
    )
    def k(y_hbm, src_hbm, dst_hbm, ew_hbm, out_hbm,
          src_v, dst_v, ew_a, ew_b, rows_v, acc, sem, sem_ea, sem_eb):
        cid = lax.axis_index("c")
        sid = lax.axis_index("s")
        wid = cid * NS + sid
        pltpu.sync_copy(src_hbm.at[wid], src_v)
        pltpu.sync_copy(dst_hbm.at[wid], dst_v)
        pltpu.async_copy(ew_hbm.at[wid, 0], ew_a, sem_ea)

        z16 = jnp.zeros((LANES,), jnp.float32)

        @pl.loop(0, CHUNK)
        def _(r):
            for s in range(d // LANES):
                rows_v[r, pl.ds(s * LANES, LANES)] = z16

        base = sid * rpt
        for b in range(nblk):
            pltpu.sync_copy(rows_v, acc.at[pl.ds(base + b * CHUNK, CHUNK)])
        plsc.subcore_barrier()

        ecb = jnp.where(cid == 0, ec0, ec1)

        @pl.loop(0, ecb, step=2)
        def _(j0):
            for half in range(2):
                j = j0 + half
                ew_cur = ew_a if half == 0 else ew_b
                ew_nxt = ew_b if half == 0 else ew_a
                se_cur = sem_ea if half == 0 else sem_eb
                se_nxt = sem_eb if half == 0 else sem_ea

                pltpu.async_copy(y_hbm.at[src_v.at[j]], rows_v, sem).wait()

                @pl.when(j + 1 < ecb)
                def _():
                    pltpu.async_copy(ew_hbm.at[wid, j + 1], ew_nxt, se_nxt)

                pltpu.make_async_copy(ew_hbm.at[wid, j], ew_cur, se_cur).wait()

                @pl.loop(0, CHUNK)
                def _(r):
                    rv = jnp.full((LANES,), r, jnp.int32)
                    ewb = plsc.load_gather(ew_cur, [rv])
                    for s in range(d // LANES):
                        sl = pl.ds(s * LANES, LANES)
                        rows_v[r, sl] = rows_v[r, sl] * ewb

                pltpu.sync_copy(rows_v, acc.at[dst_v.at[j]], add=True)

        plsc.subcore_barrier()
        for b in range(nblk):
            st = base + b * CHUNK
            pltpu.sync_copy(acc.at[pl.ds(st, CHUNK)],
                            out_hbm.at[cid, pl.ds(st, CHUNK)])

    return k(y, src_sh, dst_sh, ew_sh)


_BLK = 512


def _dot(a, b):
    return jnp.dot(a, b, preferred_element_type=jnp.float32,
                   precision=lax.Precision.HIGHEST)


def _tc_first(x_pad, w1, degp_t):
    n_pad, d = x_pad.shape

    def body(x_ref, w_ref, dp_ref, y_ref, dinv_ref):
        deg = 1.0 + dp_ref[:, 0] + dp_ref[:, 1]
        dinv = jnp.where(deg > 0, lax.rsqrt(deg), 0.0)
        y_ref[...] = dinv[:, None] * _dot(x_ref[...], w_ref[...])
        dinv_ref[...] = dinv

    return pl.pallas_call(
        body,
        grid=(n_pad // _BLK,),
        in_specs=[
            pl.BlockSpec((_BLK, d), lambda i: (i, 0)),
            pl.BlockSpec((d, d), lambda i: (0, 0)),
            pl.BlockSpec((_BLK, 2), lambda i: (i, 0)),
        ],
        out_specs=[
            pl.BlockSpec((_BLK, d), lambda i: (i, 0)),
            pl.BlockSpec((_BLK,), lambda i: (i,)),
        ],
        out_shape=[
            jax.ShapeDtypeStruct((n_pad, d), jnp.float32),
            jax.ShapeDtypeStruct((n_pad,), jnp.float32),
        ],
    )(x_pad, w1, degp_t)


def _tc_mid(pagg, y_prev, dinv, b_prev, w_next):
    n_pad, d = y_prev.shape

    def body(p_ref, y_ref, dinv_ref, b_ref, w_ref, o_ref):
        dv = dinv_ref[...]
        t = p_ref[0] + p_ref[1] + y_ref[...]
        h = jnp.maximum(dv[:, None] * t + b_ref[...], 0.0)
        o_ref[...] = dv[:, None] * _dot(h, w_ref[...])

    return pl.pallas_call(
        body,
        grid=(n_pad // _BLK,),
        in_specs=[
            pl.BlockSpec((NC, _BLK, d), lambda i: (0, i, 0)),
            pl.BlockSpec((_BLK, d), lambda i: (i, 0)),
            pl.BlockSpec((_BLK,), lambda i: (i,)),
            pl.BlockSpec((d,), lambda i: (0,)),
            pl.BlockSpec((d, d), lambda i: (0, 0)),
        ],
        out_specs=pl.BlockSpec((_BLK, d), lambda i: (i, 0)),
        out_shape=jax.ShapeDtypeStruct((n_pad, d), jnp.float32),
    )(pagg, y_prev, dinv, b_prev, w_next)


def _tc_final(pagg, y_prev, dinv, b_prev, wl, bl):
    n_pad, d = y_prev.shape
    c = wl.shape[1]

    def body(p_ref, y_ref, dinv_ref, b_ref, wl_ref, bl_ref, o_ref):
        dv = dinv_ref[...]
        t = p_ref[0] + p_ref[1] + y_ref[...]
        h = jnp.maximum(dv[:, None] * t + b_ref[...], 0.0)
        o = _dot(h, wl_ref[...]) + bl_ref[...]
        m = jnp.max(o, axis=-1, keepdims=True)
        lse = jnp.log(jnp.sum(jnp.exp(o - m), axis=-1, keepdims=True)) + m
        o_ref[...] = o - lse

    return pl.pallas_call(
        body,
        grid=(n_pad // _BLK,),
        in_specs=[
            pl.BlockSpec((NC, _BLK, d), lambda i: (0, i, 0)),
            pl.BlockSpec((_BLK, d), lambda i: (i, 0)),
            pl.BlockSpec((_BLK,), lambda i: (i,)),
            pl.BlockSpec((d,), lambda i: (0,)),
            pl.BlockSpec((d, c), lambda i: (0, 0)),
            pl.BlockSpec((c,), lambda i: (0,)),
        ],
        out_specs=pl.BlockSpec((_BLK, c), lambda i: (i, 0)),
        out_shape=jax.ShapeDtypeStruct((n_pad, c), jnp.float32),
    )(pagg, y_prev, dinv, b_prev, wl, bl)


def kernel(x, edge_index, edge_weight, W1, b1, W2, b2, W3, b3, Wl, bl):
    n, d = x.shape
    e = edge_index.shape[1]
    n_pad = -(-n // (NS * CHUNK)) * NS * CHUNK

    t = -(-e // (NS * CHUNK))
    ec0 = max(2, (int(t * 0.65) // 2) * 2)
    ec1 = -(-(t - ec0) // 2) * 2
    cap = NS * (ec0 + ec1) * CHUNK

    def shard(a):
        ap = jnp.concatenate([a, jnp.zeros((cap - e,), a.dtype)])
        c0 = ap[: NS * ec0 * CHUNK].reshape(NS, ec0, CHUNK)
        c1 = ap[NS * ec0 * CHUNK:].reshape(NS, ec1, CHUNK)
        c1 = jnp.pad(c1, ((0, 0), (0, ec0 - ec1), (0, 0)))
        return jnp.concatenate([c0, c1], axis=0)

    src = shard(edge_index[0])
    dst = shard(edge_index[1])
    ew = shard(edge_weight)
    ew_bits = lax.bitcast_convert_type(ew, jnp.int32)
    edata = jnp.stack([dst, ew_bits], axis=2)
    x_pad = jnp.pad(x, ((0, n_pad - n), (0, 0)))

    degp = _sc_deg(edata, n_pad)
    y1, dinv = _tc_first(x_pad, W1, degp.T)
    p1 = _sc_agg(y1, src, dst, ew, n_pad, ec0, ec1)
    y2 = _tc_mid(p1, y1, dinv, b1, W2)
    p2 = _sc_agg(y2, src, dst, ew, n_pad, ec0, ec1)
    y3 = _tc_mid(p2, y2, dinv, b2, W3)
    p3 = _sc_agg(y3, src, dst, ew, n_pad, ec0, ec1)
    out = _tc_final(p3, y3, dinv, b3, Wl, bl)
    return out[:n]
